# Initial kernel scaffold; baseline (speedup 1.0000x reference)
import jax
import jax.numpy as jnp
from jax.experimental import pallas as pl


def kernel(x, edge_index, edge_weight, W0, b0, W1, b1, gamma0, beta0, gamma1, beta1, act_params):
    raise NotImplementedError("write your pallas kernel here")


# speedup vs baseline: 19.5952x; 19.5952x over previous
import dataclasses
import functools
import math

import jax
import jax.numpy as jnp
from jax import lax
from jax.experimental import pallas as pl
from jax.experimental.pallas import tpu as pltpu
from jax.experimental.pallas import tpu_sc as plsc

_N = 10000
_E = 320000
_D = 128

_NC = 2
_NS = 16
_NW = _NC * _NS
_EPW = _E // _NW
_CHUNK = 80
_NCHUNK = _EPW // _CHUNK
_ACCN = 10240
_RPS = _ACCN // _NS
_ZR = 64
_G16 = _CHUNK // 16
_D16 = _D // 16

_BN_SCALE = 1.0 / math.sqrt(1.0 + 1e-5)


def _worker_id():
    return lax.axis_index("s") * _NC + lax.axis_index("c")


def _sc_compiler_params():
    cp = pltpu.CompilerParams()
    if "needs_layout_passes" in pltpu.CompilerParams.__dataclass_fields__:
        cp = dataclasses.replace(cp, needs_layout_passes=False)
    return cp


@functools.lru_cache(maxsize=None)
def _build_sc_deg():
    mesh = plsc.VectorSubcoreMesh(core_axis_name="c", subcore_axis_name="s")

    @functools.partial(
        pl.kernel,
        mesh=mesh,
        out_type=jax.ShapeDtypeStruct((_NW, 1, _N), jnp.float32),
        scratch_types=[
            pltpu.VMEM((1, _N), jnp.float32),
            pltpu.VMEM((_NCHUNK, _CHUNK), jnp.int32),
            pltpu.VMEM((_NCHUNK, _CHUNK), jnp.float32),
        ],
        compiler_params=_sc_compiler_params(),
    )
    def deg_kernel(col_hbm, ew_hbm, out_hbm, deg_v, col_v, ew_v):
        w = _worker_id()
        pltpu.sync_copy(col_hbm.at[w], col_v)
        pltpu.sync_copy(ew_hbm.at[w], ew_v)

        z16 = jnp.zeros((16,), jnp.float32)
        z16i = jnp.zeros((16,), jnp.int32)

        @pl.loop(0, _N // 16)
        def _(g):
            deg_v[0, pl.ds(g * 16, 16)] = z16

        @pl.loop(0, _NCHUNK)
        def _(ci):
            for j in range(_G16):
                idx = col_v[ci, pl.ds(j * 16, 16)]
                val = ew_v[ci, pl.ds(j * 16, 16)]
                plsc.addupdate_scatter(deg_v, [z16i, idx], val)

        pltpu.sync_copy(deg_v, out_hbm.at[w])

    return deg_kernel


_SPLAT_DNUMS = lax.GatherDimensionNumbers(
    offset_dims=(), collapsed_slice_dims=(0,), start_index_map=(0,))


def _splat(v16, t):
    idx = jnp.full((16, 1), t, jnp.int32)
    return lax.gather(v16, idx, _SPLAT_DNUMS, (1,),
                      mode=lax.GatherScatterMode.PROMISE_IN_BOUNDS)


@functools.lru_cache(maxsize=None)
def _build_sc_agg():
    mesh = plsc.VectorSubcoreMesh(core_axis_name="c", subcore_axis_name="s")

    @functools.partial(
        pl.kernel,
        mesh=mesh,
        out_type=jax.ShapeDtypeStruct((_NC, _ACCN, _D), jnp.float32),
        scratch_types=[
            pltpu.VMEM_SHARED((_ACCN, _D), jnp.float32),
            pltpu.VMEM((_N,), jnp.float32),
            pltpu.VMEM((3, _CHUNK), jnp.int32),
            pltpu.VMEM((3, _CHUNK), jnp.int32),
            pltpu.VMEM((_CHUNK, _D), jnp.float32),
            pltpu.VMEM((_CHUNK, _D), jnp.float32),
            pltpu.SemaphoreType.DMA,
            pltpu.SemaphoreType.DMA,
            pltpu.SemaphoreType.DMA,
            pltpu.SemaphoreType.DMA,
        ],
        compiler_params=_sc_compiler_params(),
    )
    def agg_kernel(h_hbm, edata_hbm, dinv_hbm, out_hbm,
                   acc, dinv_v, ebufa, ebufb, gbufa, gbufb,
                   sea, seb, sga, sgb):
        cid = lax.axis_index("c")
        sid = lax.axis_index("s")
        w = sid * _NC + cid

        pltpu.sync_copy(dinv_hbm, dinv_v)

        z16 = jnp.zeros((16,), jnp.float32)

        @pl.loop(0, _CHUNK)
        def _(r):
            for d in range(_D16):
                gbufa[r, pl.ds(d * 16, 16)] = z16

        @pl.loop(0, _RPS // _CHUNK)
        def _(k):
            pltpu.sync_copy(gbufa, acc.at[pl.ds(sid * _RPS + k * _CHUNK, _CHUNK)])

        plsc.subcore_barrier()

        def eload(ci, ebuf, sem):
            return pltpu.make_async_copy(edata_hbm.at[w, ci], ebuf, sem)

        def gather(ci, ebuf, gbuf, sem):
            del ci
            return pltpu.make_async_copy(h_hbm.at[ebuf.at[0]], gbuf, sem)

        def scale_and_scatter(ebuf, gbuf):
            @pl.loop(0, _G16)
            def _(j):
                r16 = ebuf[0, pl.ds(j * 16, 16)]
                c16 = ebuf[1, pl.ds(j * 16, 16)]
                w16 = plsc.bitcast(ebuf[2, pl.ds(j * 16, 16)], jnp.float32)
                dr = plsc.load_gather(dinv_v, [r16])
                dc = plsc.load_gather(dinv_v, [c16])
                n16 = dr * w16 * dc
                for t in range(16):
                    s16 = _splat(n16, t)
                    e = j * 16 + t
                    for d in range(_D16):
                        gbuf[e, pl.ds(d * 16, 16)] = gbuf[e, pl.ds(d * 16, 16)] * s16

            pltpu.sync_copy(gbuf, acc.at[ebuf.at[1]], add=True)

        eload(0, ebufa, sea).start()
        eload(0, ebufa, sea).wait()
        gather(0, ebufa, gbufa, sga).start()
        eload(1, ebufb, seb).start()

        def halfstep(ci, ebuf_cur, se_cur, gbuf_cur, sg_cur,
                     ebuf_nxt, se_nxt, gbuf_nxt, sg_nxt, prefetch_guard):
            eload(ci + 1, ebuf_nxt, se_nxt).wait()
            gather(ci + 1, ebuf_nxt, gbuf_nxt, sg_nxt).start()
            gather(ci, ebuf_cur, gbuf_cur, sg_cur).wait()
            scale_and_scatter(ebuf_cur, gbuf_cur)
            if prefetch_guard is None:
                eload(ci + 2, ebuf_cur, se_cur).start()
            else:
                @pl.when(prefetch_guard)
                def _():
                    eload(ci + 2, ebuf_cur, se_cur).start()

        @pl.loop(0, _NCHUNK - 1, step=2)
        def _(ci):
            halfstep(ci, ebufa, sea, gbufa, sga, ebufb, seb, gbufb, sgb, None)
            halfstep(ci + 1, ebufb, seb, gbufb, sgb, ebufa, sea, gbufa, sga,
                     ci + 3 < _NCHUNK)

        last = _NCHUNK - 1
        gather(last, ebufa, gbufa, sga).wait()
        scale_and_scatter(ebufa, gbufa)

        plsc.subcore_barrier()

        pltpu.sync_copy(acc.at[pl.ds(sid * _RPS, _RPS)],
                        out_hbm.at[cid, pl.ds(sid * _RPS, _RPS)])

    return agg_kernel


_MMB = 1000


def _mm_body(x_ref, w_ref, o_ref):
    o_ref[...] = jnp.dot(x_ref[...], w_ref[...],
                         preferred_element_type=jnp.float32)


def _tc_matmul(x, w):
    return pl.pallas_call(
        _mm_body,
        grid=(_N // _MMB,),
        in_specs=[
            pl.BlockSpec((_MMB, _D), lambda i: (i, 0)),
            pl.BlockSpec((_D, _D), lambda i: (0, 0)),
        ],
        out_specs=pl.BlockSpec((_MMB, _D), lambda i: (i, 0)),
        out_shape=jax.ShapeDtypeStruct((_N, _D), jnp.float32),
    )(x, w)


_DVB = 2000


def _dinv_body(parts_ref, o1_ref, o2_ref):
    deg = jnp.sum(parts_ref[...], axis=0, keepdims=True) + 1.0
    safe = jnp.where(deg > 0, deg, 1.0)
    dv = jnp.where(deg > 0, lax.rsqrt(safe), 0.0)
    o1_ref[...] = dv
    o2_ref[...] = jnp.transpose(dv * dv)


def _tc_dinv(parts):
    return pl.pallas_call(
        _dinv_body,
        out_shape=[
            jax.ShapeDtypeStruct((1, _N), jnp.float32),
            jax.ShapeDtypeStruct((_N, 1), jnp.float32),
        ],
    )(parts)


def _post_body(p0_ref, p1_ref, h_ref, d2_ref, b_ref, g_ref, be_ref, al_ref,
               o_ref, *, apply_act):
    agg = (p0_ref[...] + p1_ref[...] + d2_ref[...] * h_ref[...]) + b_ref[...]
    hbn = agg * (g_ref[...] * _BN_SCALE) + be_ref[...]
    if apply_act:
        alpha = al_ref[0, 0]
        gelu = hbn * 0.5 * (1.0 + lax.erf(hbn * (1.0 / math.sqrt(2.0))))
        hbn = alpha * jnp.maximum(hbn, 0.0) + (1.0 - alpha) * gelu
    o_ref[...] = hbn


def _tc_post(p0, p1, h, d2, b, g, be, alpha, apply_act):
    row_spec = pl.BlockSpec((_MMB, _D), lambda i: (i, 0))
    vec_spec = pl.BlockSpec((1, _D), lambda i: (0, 0))
    return pl.pallas_call(
        functools.partial(_post_body, apply_act=apply_act),
        grid=(_N // _MMB,),
        in_specs=[
            row_spec, row_spec, row_spec,
            pl.BlockSpec((_MMB, 1), lambda i: (i, 0)),
            vec_spec, vec_spec, vec_spec,
            pl.BlockSpec(memory_space=pltpu.SMEM),
        ],
        out_specs=row_spec,
        out_shape=jax.ShapeDtypeStruct((_N, _D), jnp.float32),
    )(p0, p1, h, d2, b, g, be, alpha)


def kernel(x, edge_index, edge_weight, W0, b0, W1, b1,
           gamma0, beta0, gamma1, beta1, act_params):
    row2 = edge_index[0].reshape(_NW, _NCHUNK, _CHUNK)
    col2 = edge_index[1].reshape(_NW, _NCHUNK, _CHUNK)
    ewi = lax.bitcast_convert_type(edge_weight, jnp.int32)
    ew2i = ewi.reshape(_NW, _NCHUNK, _CHUNK)
    edata = jnp.stack([row2, col2, ew2i], axis=2)
    ew2 = edge_weight.reshape(_NW, _NCHUNK, _CHUNK)

    deg_parts = _build_sc_deg()(col2, ew2).reshape(_NW, _N)
    dinv_row, dinv2_col = _tc_dinv(deg_parts)
    dinv = dinv_row.reshape(_N)

    alpha = jax.nn.sigmoid(act_params[0]).reshape(1, 1)

    b0r = b0.reshape(1, _D)
    g0r = gamma0.reshape(1, _D)
    be0r = beta0.reshape(1, _D)
    b1r = b1.reshape(1, _D)
    g1r = gamma1.reshape(1, _D)
    be1r = beta1.reshape(1, _D)

    agg = _build_sc_agg()

    h0 = _tc_matmul(x, W0)
    parts0 = agg(h0, edata, dinv)
    y0 = _tc_post(parts0[0, :_N], parts0[1, :_N], h0, dinv2_col,
                  b0r, g0r, be0r, alpha, True)

    h1 = _tc_matmul(y0, W1)
    parts1 = agg(h1, edata, dinv)
    y1 = _tc_post(parts1[0, :_N], parts1[1, :_N], h1, dinv2_col,
                  b1r, g1r, be1r, alpha, False)
    return y1


# speedup vs baseline: 23.1708x; 1.1825x over previous
import dataclasses
import functools
import math

import jax
import jax.numpy as jnp
from jax import lax
from jax.experimental import pallas as pl
from jax.experimental.pallas import tpu as pltpu
from jax.experimental.pallas import tpu_sc as plsc

_N = 10000
_E = 320000
_D = 128

_NC = 2
_NS = 16
_NW = _NC * _NS
_EPW = _E // _NW
_CHUNK = 80
_NCHUNK = _EPW // _CHUNK
_ACCN = 10240
_RPS = _ACCN // _NS
_ZR = 64
_G16 = _CHUNK // 16
_D16 = _D // 16

_BN_SCALE = 1.0 / math.sqrt(1.0 + 1e-5)


def _worker_id():
    return lax.axis_index("s") * _NC + lax.axis_index("c")


def _sc_compiler_params():
    cp = pltpu.CompilerParams()
    if "needs_layout_passes" in pltpu.CompilerParams.__dataclass_fields__:
        cp = dataclasses.replace(cp, needs_layout_passes=False)
    return cp


@functools.lru_cache(maxsize=None)
def _build_sc_deg():
    mesh = plsc.VectorSubcoreMesh(core_axis_name="c", subcore_axis_name="s")

    @functools.partial(
        pl.kernel,
        mesh=mesh,
        out_type=jax.ShapeDtypeStruct((_NW, 1, _N), jnp.float32),
        scratch_types=[
            pltpu.VMEM((1, _N), jnp.float32),
            pltpu.VMEM((_NCHUNK, _CHUNK), jnp.int32),
            pltpu.VMEM((_NCHUNK, _CHUNK), jnp.float32),
        ],
        compiler_params=_sc_compiler_params(),
    )
    def deg_kernel(col_hbm, ew_hbm, out_hbm, deg_v, col_v, ew_v):
        w = _worker_id()
        pltpu.sync_copy(col_hbm.at[w], col_v)
        pltpu.sync_copy(ew_hbm.at[w], ew_v)

        z16 = jnp.zeros((16,), jnp.float32)
        z16i = jnp.zeros((16,), jnp.int32)

        @pl.loop(0, _N // 16)
        def _(g):
            deg_v[0, pl.ds(g * 16, 16)] = z16

        @pl.loop(0, _NCHUNK)
        def _(ci):
            for j in range(_G16):
                idx = col_v[ci, pl.ds(j * 16, 16)]
                val = ew_v[ci, pl.ds(j * 16, 16)]
                plsc.addupdate_scatter(deg_v, [z16i, idx], val)

        pltpu.sync_copy(deg_v, out_hbm.at[w])

    return deg_kernel


@functools.lru_cache(maxsize=None)
def _build_sc_norm():
    mesh = plsc.VectorSubcoreMesh(core_axis_name="c", subcore_axis_name="s")

    @functools.partial(
        pl.kernel,
        mesh=mesh,
        out_type=jax.ShapeDtypeStruct((_NW, _NCHUNK, _CHUNK), jnp.float32),
        scratch_types=[
            pltpu.VMEM((_NCHUNK, _CHUNK), jnp.int32),
            pltpu.VMEM((_NCHUNK, _CHUNK), jnp.int32),
            pltpu.VMEM((_NCHUNK, _CHUNK), jnp.float32),
            pltpu.VMEM((_N,), jnp.float32),
        ],
        compiler_params=_sc_compiler_params(),
    )
    def norm_kernel(row_hbm, col_hbm, ew_hbm, dinv_hbm, out_hbm,
                    row_v, col_v, ew_v, dinv_v):
        w = _worker_id()
        pltpu.sync_copy(dinv_hbm, dinv_v)
        pltpu.sync_copy(row_hbm.at[w], row_v)
        pltpu.sync_copy(col_hbm.at[w], col_v)
        pltpu.sync_copy(ew_hbm.at[w], ew_v)

        @pl.loop(0, _NCHUNK)
        def _(ci):
            for j in range(_G16):
                r16 = row_v[ci, pl.ds(j * 16, 16)]
                c16 = col_v[ci, pl.ds(j * 16, 16)]
                w16 = ew_v[ci, pl.ds(j * 16, 16)]
                dr = plsc.load_gather(dinv_v, [r16])
                dc = plsc.load_gather(dinv_v, [c16])
                ew_v[ci, pl.ds(j * 16, 16)] = dr * w16 * dc

        pltpu.sync_copy(ew_v, out_hbm.at[w])

    return norm_kernel


_SPLAT_DNUMS = lax.GatherDimensionNumbers(
    offset_dims=(), collapsed_slice_dims=(0,), start_index_map=(0,))


def _splat(v16, t):
    idx = jnp.full((16, 1), t, jnp.int32)
    return lax.gather(v16, idx, _SPLAT_DNUMS, (1,),
                      mode=lax.GatherScatterMode.PROMISE_IN_BOUNDS)


@functools.lru_cache(maxsize=None)
def _build_sc_agg():
    mesh = plsc.VectorSubcoreMesh(core_axis_name="c", subcore_axis_name="s")

    @functools.partial(
        pl.kernel,
        mesh=mesh,
        out_type=jax.ShapeDtypeStruct((_NC, _ACCN, _D), jnp.float32),
        scratch_types=[
            pltpu.VMEM_SHARED((_ACCN, _D), jnp.float32),
            pltpu.VMEM((3, _CHUNK), jnp.int32),
            pltpu.VMEM((3, _CHUNK), jnp.int32),
            pltpu.VMEM((_CHUNK,), jnp.int32),
            pltpu.VMEM((_CHUNK,), jnp.int32),
            pltpu.VMEM((_CHUNK, _D), jnp.float32),
            pltpu.VMEM((_CHUNK, _D), jnp.float32),
            pltpu.VMEM((_CHUNK, _D), jnp.float32),
            pltpu.VMEM((_CHUNK, _D), jnp.float32),
            pltpu.SemaphoreType.DMA,
            pltpu.SemaphoreType.DMA,
            pltpu.SemaphoreType.DMA,
            pltpu.SemaphoreType.DMA,
            pltpu.SemaphoreType.DMA,
            pltpu.SemaphoreType.DMA,
        ],
        compiler_params=_sc_compiler_params(),
    )
    def agg_kernel(h_hbm, edata_hbm, out_hbm,
                   acc, ebufa, ebufb, cbufa, cbufb, gbufa, gbufb,
                   sbufa, sbufb, sea, seb, sga, sgb, ssa, ssb):
        cid = lax.axis_index("c")
        sid = lax.axis_index("s")
        w = sid * _NC + cid

        z16 = jnp.zeros((16,), jnp.float32)

        @pl.loop(0, _CHUNK)
        def _(r):
            for d in range(_D16):
                gbufa[r, pl.ds(d * 16, 16)] = z16

        @pl.loop(0, _RPS // _CHUNK)
        def _(k):
            pltpu.sync_copy(gbufa, acc.at[pl.ds(sid * _RPS + k * _CHUNK, _CHUNK)])

        plsc.subcore_barrier()

        def eload(ci, ebuf, sem):
            return pltpu.make_async_copy(edata_hbm.at[w, ci], ebuf, sem)

        def gather(ebuf, gbuf, sem):
            return pltpu.make_async_copy(h_hbm.at[ebuf.at[0]], gbuf, sem)

        def scatter_wait(sbuf, cbuf, sem):
            pltpu.make_async_copy(sbuf, acc.at[cbuf], sem).wait()

        def scale(ebuf, cbuf, gbuf, sbuf):
            @pl.loop(0, _G16)
            def _(j):
                cbuf[pl.ds(j * 16, 16)] = ebuf[1, pl.ds(j * 16, 16)]
                n16 = plsc.bitcast(ebuf[2, pl.ds(j * 16, 16)], jnp.float32)
                for t in range(16):
                    s16 = _splat(n16, t)
                    e = j * 16 + t
                    for d in range(_D16):
                        sbuf[e, pl.ds(d * 16, 16)] = gbuf[e, pl.ds(d * 16, 16)] * s16

        eload(0, ebufa, sea).start()
        eload(0, ebufa, sea).wait()
        gather(ebufa, gbufa, sga).start()
        eload(1, ebufb, seb).start()

        @pl.loop(0, _NCHUNK - 1, step=2)
        def _(ci):
            eload(ci + 1, ebufb, seb).wait()
            gather(ebufb, gbufb, sgb).start()
            gather(ebufa, gbufa, sga).wait()

            @pl.when(ci >= 2)
            def _():
                scatter_wait(sbufa, cbufa, ssa)

            scale(ebufa, cbufa, gbufa, sbufa)
            pltpu.async_copy(sbufa, acc.at[cbufa], ssa, add=True)
            eload(ci + 2, ebufa, sea).start()

            eload(ci + 2, ebufa, sea).wait()
            gather(ebufa, gbufa, sga).start()
            gather(ebufb, gbufb, sgb).wait()

            @pl.when(ci >= 2)
            def _():
                scatter_wait(sbufb, cbufb, ssb)

            scale(ebufb, cbufb, gbufb, sbufb)
            pltpu.async_copy(sbufb, acc.at[cbufb], ssb, add=True)

            @pl.when(ci + 3 < _NCHUNK)
            def _():
                eload(ci + 3, ebufb, seb).start()

        gather(ebufa, gbufa, sga).wait()
        scatter_wait(sbufa, cbufa, ssa)
        scale(ebufa, cbufa, gbufa, sbufa)
        pltpu.async_copy(sbufa, acc.at[cbufa], ssa, add=True)
        scatter_wait(sbufb, cbufb, ssb)
        scatter_wait(sbufa, cbufa, ssa)

        plsc.subcore_barrier()

        pltpu.sync_copy(acc.at[pl.ds(sid * _RPS, _RPS)],
                        out_hbm.at[cid, pl.ds(sid * _RPS, _RPS)])

    return agg_kernel


_MMB = 1000


def _mm_body(x_ref, w_ref, o_ref):
    o_ref[...] = jnp.dot(x_ref[...], w_ref[...],
                         preferred_element_type=jnp.float32)


def _tc_matmul(x, w):
    return pl.pallas_call(
        _mm_body,
        grid=(_N // _MMB,),
        in_specs=[
            pl.BlockSpec((_MMB, _D), lambda i: (i, 0)),
            pl.BlockSpec((_D, _D), lambda i: (0, 0)),
        ],
        out_specs=pl.BlockSpec((_MMB, _D), lambda i: (i, 0)),
        out_shape=jax.ShapeDtypeStruct((_N, _D), jnp.float32),
    )(x, w)


_DVB = 2000


def _dinv_body(parts_ref, o1_ref, o2_ref):
    deg = jnp.sum(parts_ref[...], axis=0, keepdims=True) + 1.0
    safe = jnp.where(deg > 0, deg, 1.0)
    dv = jnp.where(deg > 0, lax.rsqrt(safe), 0.0)
    o1_ref[...] = dv
    o2_ref[...] = jnp.transpose(dv * dv)


def _tc_dinv(parts):
    return pl.pallas_call(
        _dinv_body,
        out_shape=[
            jax.ShapeDtypeStruct((1, _N), jnp.float32),
            jax.ShapeDtypeStruct((_N, 1), jnp.float32),
        ],
    )(parts)


def _post_body(p0_ref, p1_ref, h_ref, d2_ref, b_ref, g_ref, be_ref, al_ref,
               o_ref, *, apply_act):
    agg = (p0_ref[...] + p1_ref[...] + d2_ref[...] * h_ref[...]) + b_ref[...]
    hbn = agg * (g_ref[...] * _BN_SCALE) + be_ref[...]
    if apply_act:
        alpha = al_ref[0, 0]
        gelu = hbn * 0.5 * (1.0 + lax.erf(hbn * (1.0 / math.sqrt(2.0))))
        hbn = alpha * jnp.maximum(hbn, 0.0) + (1.0 - alpha) * gelu
    o_ref[...] = hbn


def _tc_post(p0, p1, h, d2, b, g, be, alpha, apply_act):
    row_spec = pl.BlockSpec((_MMB, _D), lambda i: (i, 0))
    vec_spec = pl.BlockSpec((1, _D), lambda i: (0, 0))
    return pl.pallas_call(
        functools.partial(_post_body, apply_act=apply_act),
        grid=(_N // _MMB,),
        in_specs=[
            row_spec, row_spec, row_spec,
            pl.BlockSpec((_MMB, 1), lambda i: (i, 0)),
            vec_spec, vec_spec, vec_spec,
            pl.BlockSpec(memory_space=pltpu.SMEM),
        ],
        out_specs=row_spec,
        out_shape=jax.ShapeDtypeStruct((_N, _D), jnp.float32),
    )(p0, p1, h, d2, b, g, be, alpha)


def kernel(x, edge_index, edge_weight, W0, b0, W1, b1,
           gamma0, beta0, gamma1, beta1, act_params):
    row2 = edge_index[0].reshape(_NW, _NCHUNK, _CHUNK)
    col2 = edge_index[1].reshape(_NW, _NCHUNK, _CHUNK)
    ew2 = edge_weight.reshape(_NW, _NCHUNK, _CHUNK)

    deg_parts = _build_sc_deg()(col2, ew2).reshape(_NW, _N)
    dinv_row, dinv2_col = _tc_dinv(deg_parts)
    dinv = dinv_row.reshape(_N)
    norm2 = _build_sc_norm()(row2, col2, ew2, dinv)
    normi = lax.bitcast_convert_type(norm2, jnp.int32)
    edata = jnp.stack([row2, col2, normi], axis=2)

    alpha = jax.nn.sigmoid(act_params[0]).reshape(1, 1)

    b0r = b0.reshape(1, _D)
    g0r = gamma0.reshape(1, _D)
    be0r = beta0.reshape(1, _D)
    b1r = b1.reshape(1, _D)
    g1r = gamma1.reshape(1, _D)
    be1r = beta1.reshape(1, _D)

    agg = _build_sc_agg()

    h0 = _tc_matmul(x, W0)
    parts0 = agg(h0, edata)
    y0 = _tc_post(parts0[0, :_N], parts0[1, :_N], h0, dinv2_col,
                  b0r, g0r, be0r, alpha, True)

    h1 = _tc_matmul(y0, W1)
    parts1 = agg(h1, edata)
    y1 = _tc_post(parts1[0, :_N], parts1[1, :_N], h1, dinv2_col,
                  b1r, g1r, be1r, alpha, False)
    return y1


# speedup vs baseline: 24.0507x; 1.0380x over previous
import dataclasses
import functools
import math

import jax
import jax.numpy as jnp
from jax import lax
from jax.experimental import pallas as pl
from jax.experimental.pallas import tpu as pltpu
from jax.experimental.pallas import tpu_sc as plsc

_N = 10000
_E = 320000
_D = 128

_NC = 2
_NS = 16
_NW = _NC * _NS
_EPW = _E // _NW
_CHUNK = 80
_NCHUNK = _EPW // _CHUNK
_ACCN = 10240
_RPS = _ACCN // _NS
_ZR = 64
_G16 = _CHUNK // 16
_D16 = _D // 16

_BN_SCALE = 1.0 / math.sqrt(1.0 + 1e-5)


def _worker_id():
    return lax.axis_index("s") * _NC + lax.axis_index("c")


def _sc_compiler_params():
    cp = pltpu.CompilerParams()
    if "needs_layout_passes" in pltpu.CompilerParams.__dataclass_fields__:
        cp = dataclasses.replace(cp, needs_layout_passes=False)
    return cp


@functools.lru_cache(maxsize=None)
def _build_sc_deg():
    mesh = plsc.VectorSubcoreMesh(core_axis_name="c", subcore_axis_name="s")

    @functools.partial(
        pl.kernel,
        mesh=mesh,
        out_type=jax.ShapeDtypeStruct((_NW, 1, _N), jnp.float32),
        scratch_types=[
            pltpu.VMEM((1, _N), jnp.float32),
            pltpu.VMEM((_NCHUNK, _CHUNK), jnp.int32),
            pltpu.VMEM((_NCHUNK, _CHUNK), jnp.float32),
        ],
        compiler_params=_sc_compiler_params(),
    )
    def deg_kernel(col_hbm, ew_hbm, out_hbm, deg_v, col_v, ew_v):
        w = _worker_id()
        pltpu.sync_copy(col_hbm.at[w], col_v)
        pltpu.sync_copy(ew_hbm.at[w], ew_v)

        z16 = jnp.zeros((16,), jnp.float32)
        z16i = jnp.zeros((16,), jnp.int32)

        @pl.loop(0, _N // 16)
        def _(g):
            deg_v[0, pl.ds(g * 16, 16)] = z16

        @pl.loop(0, _NCHUNK)
        def _(ci):
            for j in range(_G16):
                idx = col_v[ci, pl.ds(j * 16, 16)]
                val = ew_v[ci, pl.ds(j * 16, 16)]
                plsc.addupdate_scatter(deg_v, [z16i, idx], val)

        pltpu.sync_copy(deg_v, out_hbm.at[w])

    return deg_kernel


@functools.lru_cache(maxsize=None)
def _build_sc_norm():
    mesh = plsc.VectorSubcoreMesh(core_axis_name="c", subcore_axis_name="s")

    @functools.partial(
        pl.kernel,
        mesh=mesh,
        out_type=jax.ShapeDtypeStruct((_NW, _NCHUNK, _CHUNK), jnp.float32),
        scratch_types=[
            pltpu.VMEM((_NCHUNK, _CHUNK), jnp.int32),
            pltpu.VMEM((_NCHUNK, _CHUNK), jnp.int32),
            pltpu.VMEM((_NCHUNK, _CHUNK), jnp.float32),
            pltpu.VMEM((_N,), jnp.float32),
        ],
        compiler_params=_sc_compiler_params(),
    )
    def norm_kernel(row_hbm, col_hbm, ew_hbm, dinv_hbm, out_hbm,
                    row_v, col_v, ew_v, dinv_v):
        w = _worker_id()
        pltpu.sync_copy(dinv_hbm, dinv_v)
        pltpu.sync_copy(row_hbm.at[w], row_v)
        pltpu.sync_copy(col_hbm.at[w], col_v)
        pltpu.sync_copy(ew_hbm.at[w], ew_v)

        @pl.loop(0, _NCHUNK)
        def _(ci):
            for j in range(_G16):
                r16 = row_v[ci, pl.ds(j * 16, 16)]
                c16 = col_v[ci, pl.ds(j * 16, 16)]
                w16 = ew_v[ci, pl.ds(j * 16, 16)]
                dr = plsc.load_gather(dinv_v, [r16])
                dc = plsc.load_gather(dinv_v, [c16])
                ew_v[ci, pl.ds(j * 16, 16)] = dr * w16 * dc

        pltpu.sync_copy(ew_v, out_hbm.at[w])

    return norm_kernel


_SPLAT_DNUMS = lax.GatherDimensionNumbers(
    offset_dims=(), collapsed_slice_dims=(0,), start_index_map=(0,))


def _splat(v16, t):
    idx = jnp.full((16, 1), t, jnp.int32)
    return lax.gather(v16, idx, _SPLAT_DNUMS, (1,),
                      mode=lax.GatherScatterMode.PROMISE_IN_BOUNDS)


@functools.lru_cache(maxsize=None)
def _build_sc_agg():
    mesh = plsc.VectorSubcoreMesh(core_axis_name="c", subcore_axis_name="s")

    @functools.partial(
        pl.kernel,
        mesh=mesh,
        out_type=jax.ShapeDtypeStruct((_NC, _ACCN, _D), jnp.float32),
        scratch_types=[
            pltpu.VMEM_SHARED((_ACCN, _D), jnp.float32),
            pltpu.VMEM((3, _CHUNK), jnp.int32),
            pltpu.VMEM((3, _CHUNK), jnp.int32),
            pltpu.VMEM((_CHUNK,), jnp.int32),
            pltpu.VMEM((_CHUNK,), jnp.int32),
            pltpu.VMEM((_CHUNK,), jnp.float32),
            pltpu.VMEM((_CHUNK,), jnp.float32),
            pltpu.VMEM((_CHUNK, _D), jnp.float32),
            pltpu.VMEM((_CHUNK, _D), jnp.float32),
            pltpu.VMEM((_CHUNK, _D), jnp.float32),
            pltpu.VMEM((_CHUNK, _D), jnp.float32),
            pltpu.SemaphoreType.DMA,
            pltpu.SemaphoreType.DMA,
            pltpu.SemaphoreType.DMA,
            pltpu.SemaphoreType.DMA,
            pltpu.SemaphoreType.DMA,
            pltpu.SemaphoreType.DMA,
        ],
        compiler_params=_sc_compiler_params(),
    )
    def agg_kernel(h_hbm, edata_hbm, out_hbm,
                   acc, ebufa, ebufb, cbufa, cbufb, nbufa, nbufb,
                   gbufa, gbufb, sbufa, sbufb, sea, seb, sga, sgb, ssa, ssb):
        cid = lax.axis_index("c")
        sid = lax.axis_index("s")
        w = sid * _NC + cid

        z16 = jnp.zeros((16,), jnp.float32)

        @pl.loop(0, _CHUNK)
        def _(r):
            for d in range(_D16):
                gbufa[r, pl.ds(d * 16, 16)] = z16

        @pl.loop(0, _RPS // _CHUNK)
        def _(k):
            pltpu.sync_copy(gbufa, acc.at[pl.ds(sid * _RPS + k * _CHUNK, _CHUNK)])

        plsc.subcore_barrier()

        def eload(ci, ebuf, sem):
            return pltpu.make_async_copy(edata_hbm.at[w, ci], ebuf, sem)

        def gather(ebuf, gbuf, sem):
            return pltpu.make_async_copy(h_hbm.at[ebuf.at[0]], gbuf, sem)

        def scatter_wait(sbuf, cbuf, sem):
            pltpu.make_async_copy(sbuf, acc.at[cbuf], sem).wait()

        def stage_edge(ebuf, cbuf, nbuf):
            @pl.loop(0, _G16)
            def _(j):
                cbuf[pl.ds(j * 16, 16)] = ebuf[1, pl.ds(j * 16, 16)]
                nbuf[pl.ds(j * 16, 16)] = plsc.bitcast(
                    ebuf[2, pl.ds(j * 16, 16)], jnp.float32)

        def scale(nbuf, gbuf, sbuf):
            @pl.loop(0, _G16)
            def _(j):
                n16 = nbuf[pl.ds(j * 16, 16)]
                for t in range(16):
                    s16 = _splat(n16, t)
                    e = j * 16 + t
                    for d in range(_D16):
                        sbuf[e, pl.ds(d * 16, 16)] = gbuf[e, pl.ds(d * 16, 16)] * s16

        eload(0, ebufa, sea).start()
        eload(0, ebufa, sea).wait()
        gather(ebufa, gbufa, sga).start()
        eload(1, ebufb, seb).start()

        @pl.loop(0, _NCHUNK - 1, step=2)
        def _(ci):
            eload(ci + 1, ebufb, seb).wait()
            gather(ebufb, gbufb, sgb).start()
            gather(ebufa, gbufa, sga).wait()

            @pl.when(ci >= 2)
            def _():
                scatter_wait(sbufa, cbufa, ssa)

            stage_edge(ebufa, cbufa, nbufa)
            eload(ci + 2, ebufa, sea).start()
            scale(nbufa, gbufa, sbufa)
            pltpu.async_copy(sbufa, acc.at[cbufa], ssa, add=True)

            eload(ci + 2, ebufa, sea).wait()
            gather(ebufa, gbufa, sga).start()
            gather(ebufb, gbufb, sgb).wait()

            @pl.when(ci >= 2)
            def _():
                scatter_wait(sbufb, cbufb, ssb)

            stage_edge(ebufb, cbufb, nbufb)

            @pl.when(ci + 3 < _NCHUNK)
            def _():
                eload(ci + 3, ebufb, seb).start()

            scale(nbufb, gbufb, sbufb)
            pltpu.async_copy(sbufb, acc.at[cbufb], ssb, add=True)

        gather(ebufa, gbufa, sga).wait()
        scatter_wait(sbufa, cbufa, ssa)
        stage_edge(ebufa, cbufa, nbufa)
        scale(nbufa, gbufa, sbufa)
        pltpu.async_copy(sbufa, acc.at[cbufa], ssa, add=True)
        scatter_wait(sbufb, cbufb, ssb)
        scatter_wait(sbufa, cbufa, ssa)

        plsc.subcore_barrier()

        pltpu.sync_copy(acc.at[pl.ds(sid * _RPS, _RPS)],
                        out_hbm.at[cid, pl.ds(sid * _RPS, _RPS)])

    return agg_kernel


_MMB = 1000


def _mm_body(x_ref, w_ref, o_ref):
    o_ref[...] = jnp.dot(x_ref[...], w_ref[...],
                         preferred_element_type=jnp.float32)


def _tc_matmul(x, w):
    return pl.pallas_call(
        _mm_body,
        grid=(_N // _MMB,),
        in_specs=[
            pl.BlockSpec((_MMB, _D), lambda i: (i, 0)),
            pl.BlockSpec((_D, _D), lambda i: (0, 0)),
        ],
        out_specs=pl.BlockSpec((_MMB, _D), lambda i: (i, 0)),
        out_shape=jax.ShapeDtypeStruct((_N, _D), jnp.float32),
    )(x, w)


_DVB = 2000


def _dinv_body(parts_ref, o1_ref, o2_ref):
    deg = jnp.sum(parts_ref[...], axis=0, keepdims=True) + 1.0
    safe = jnp.where(deg > 0, deg, 1.0)
    dv = jnp.where(deg > 0, lax.rsqrt(safe), 0.0)
    o1_ref[...] = dv
    o2_ref[...] = jnp.transpose(dv * dv)


def _tc_dinv(parts):
    return pl.pallas_call(
        _dinv_body,
        out_shape=[
            jax.ShapeDtypeStruct((1, _N), jnp.float32),
            jax.ShapeDtypeStruct((_N, 1), jnp.float32),
        ],
    )(parts)


def _post_body(p0_ref, p1_ref, h_ref, d2_ref, b_ref, g_ref, be_ref, al_ref,
               o_ref, *, apply_act):
    agg = (p0_ref[...] + p1_ref[...] + d2_ref[...] * h_ref[...]) + b_ref[...]
    hbn = agg * (g_ref[...] * _BN_SCALE) + be_ref[...]
    if apply_act:
        alpha = al_ref[0, 0]
        gelu = hbn * 0.5 * (1.0 + lax.erf(hbn * (1.0 / math.sqrt(2.0))))
        hbn = alpha * jnp.maximum(hbn, 0.0) + (1.0 - alpha) * gelu
    o_ref[...] = hbn


def _tc_post(p0, p1, h, d2, b, g, be, alpha, apply_act):
    row_spec = pl.BlockSpec((_MMB, _D), lambda i: (i, 0))
    vec_spec = pl.BlockSpec((1, _D), lambda i: (0, 0))
    return pl.pallas_call(
        functools.partial(_post_body, apply_act=apply_act),
        grid=(_N // _MMB,),
        in_specs=[
            row_spec, row_spec, row_spec,
            pl.BlockSpec((_MMB, 1), lambda i: (i, 0)),
            vec_spec, vec_spec, vec_spec,
            pl.BlockSpec(memory_space=pltpu.SMEM),
        ],
        out_specs=row_spec,
        out_shape=jax.ShapeDtypeStruct((_N, _D), jnp.float32),
    )(p0, p1, h, d2, b, g, be, alpha)


def kernel(x, edge_index, edge_weight, W0, b0, W1, b1,
           gamma0, beta0, gamma1, beta1, act_params):
    row2 = edge_index[0].reshape(_NW, _NCHUNK, _CHUNK)
    col2 = edge_index[1].reshape(_NW, _NCHUNK, _CHUNK)
    ew2 = edge_weight.reshape(_NW, _NCHUNK, _CHUNK)

    deg_parts = _build_sc_deg()(col2, ew2).reshape(_NW, _N)
    dinv_row, dinv2_col = _tc_dinv(deg_parts)
    dinv = dinv_row.reshape(_N)
    norm2 = _build_sc_norm()(row2, col2, ew2, dinv)
    normi = lax.bitcast_convert_type(norm2, jnp.int32)
    edata = jnp.stack([row2, col2, normi], axis=2)

    alpha = jax.nn.sigmoid(act_params[0]).reshape(1, 1)

    b0r = b0.reshape(1, _D)
    g0r = gamma0.reshape(1, _D)
    be0r = beta0.reshape(1, _D)
    b1r = b1.reshape(1, _D)
    g1r = gamma1.reshape(1, _D)
    be1r = beta1.reshape(1, _D)

    agg = _build_sc_agg()

    h0 = _tc_matmul(x, W0)
    parts0 = agg(h0, edata)
    y0 = _tc_post(parts0[0, :_N], parts0[1, :_N], h0, dinv2_col,
                  b0r, g0r, be0r, alpha, True)

    h1 = _tc_matmul(y0, W1)
    parts1 = agg(h1, edata)
    y1 = _tc_post(parts1[0, :_N], parts1[1, :_N], h1, dinv2_col,
                  b1r, g1r, be1r, alpha, False)
    return y1


# speedup vs baseline: 25.3485x; 1.0540x over previous
import dataclasses
import functools
import math

import jax
import jax.numpy as jnp
from jax import lax
from jax.experimental import pallas as pl
from jax.experimental.pallas import tpu as pltpu
from jax.experimental.pallas import tpu_sc as plsc

_N = 10000
_E = 320000
_D = 128

_NC = 2
_NS = 16
_NW = _NC * _NS
_EPW = _E // _NW
_CHUNK = 80
_NCHUNK = _EPW // _CHUNK
_ACCN = 10240
_RPS = _ACCN // _NS
_ZR = 64
_G16 = _CHUNK // 16
_D16 = _D // 16

_BN_SCALE = 1.0 / math.sqrt(1.0 + 1e-5)


def _worker_id():
    return lax.axis_index("s") * _NC + lax.axis_index("c")


def _sc_compiler_params():
    cp = pltpu.CompilerParams()
    if "needs_layout_passes" in pltpu.CompilerParams.__dataclass_fields__:
        cp = dataclasses.replace(cp, needs_layout_passes=False)
    return cp


@functools.lru_cache(maxsize=None)
def _build_sc_deg():
    mesh = plsc.VectorSubcoreMesh(core_axis_name="c", subcore_axis_name="s")

    @functools.partial(
        pl.kernel,
        mesh=mesh,
        out_type=jax.ShapeDtypeStruct((_NW, 1, _N), jnp.float32),
        scratch_types=[
            pltpu.VMEM((1, _N), jnp.float32),
            pltpu.VMEM((_NCHUNK, _CHUNK), jnp.int32),
            pltpu.VMEM((_NCHUNK, _CHUNK), jnp.float32),
        ],
        compiler_params=_sc_compiler_params(),
    )
    def deg_kernel(col_hbm, ew_hbm, out_hbm, deg_v, col_v, ew_v):
        w = _worker_id()
        pltpu.sync_copy(col_hbm.at[w], col_v)
        pltpu.sync_copy(ew_hbm.at[w], ew_v)

        z16 = jnp.zeros((16,), jnp.float32)
        z16i = jnp.zeros((16,), jnp.int32)

        @pl.loop(0, _N // 16)
        def _(g):
            deg_v[0, pl.ds(g * 16, 16)] = z16

        @pl.loop(0, _NCHUNK)
        def _(ci):
            for j in range(_G16):
                idx = col_v[ci, pl.ds(j * 16, 16)]
                val = ew_v[ci, pl.ds(j * 16, 16)]
                plsc.addupdate_scatter(deg_v, [z16i, idx], val)

        pltpu.sync_copy(deg_v, out_hbm.at[w])

    return deg_kernel


@functools.lru_cache(maxsize=None)
def _build_sc_norm():
    mesh = plsc.VectorSubcoreMesh(core_axis_name="c", subcore_axis_name="s")

    @functools.partial(
        pl.kernel,
        mesh=mesh,
        out_type=jax.ShapeDtypeStruct((_NW, _NCHUNK, _CHUNK), jnp.float32),
        scratch_types=[
            pltpu.VMEM((_NCHUNK, _CHUNK), jnp.int32),
            pltpu.VMEM((_NCHUNK, _CHUNK), jnp.int32),
            pltpu.VMEM((_NCHUNK, _CHUNK), jnp.float32),
            pltpu.VMEM((_N,), jnp.float32),
        ],
        compiler_params=_sc_compiler_params(),
    )
    def norm_kernel(row_hbm, col_hbm, ew_hbm, dinv_hbm, out_hbm,
                    row_v, col_v, ew_v, dinv_v):
        w = _worker_id()
        pltpu.sync_copy(dinv_hbm, dinv_v)
        pltpu.sync_copy(row_hbm.at[w], row_v)
        pltpu.sync_copy(col_hbm.at[w], col_v)
        pltpu.sync_copy(ew_hbm.at[w], ew_v)

        @pl.loop(0, _NCHUNK)
        def _(ci):
            for j in range(_G16):
                r16 = row_v[ci, pl.ds(j * 16, 16)]
                c16 = col_v[ci, pl.ds(j * 16, 16)]
                w16 = ew_v[ci, pl.ds(j * 16, 16)]
                dr = plsc.load_gather(dinv_v, [r16])
                dc = plsc.load_gather(dinv_v, [c16])
                ew_v[ci, pl.ds(j * 16, 16)] = dr * w16 * dc

        pltpu.sync_copy(ew_v, out_hbm.at[w])

    return norm_kernel


_SPLAT_DNUMS = lax.GatherDimensionNumbers(
    offset_dims=(), collapsed_slice_dims=(0,), start_index_map=(0,))


def _splat(v16, t):
    idx = jnp.full((16, 1), t, jnp.int32)
    return lax.gather(v16, idx, _SPLAT_DNUMS, (1,),
                      mode=lax.GatherScatterMode.PROMISE_IN_BOUNDS)


@functools.lru_cache(maxsize=None)
def _build_sc_agg():
    mesh = plsc.VectorSubcoreMesh(core_axis_name="c", subcore_axis_name="s")

    @functools.partial(
        pl.kernel,
        mesh=mesh,
        out_type=jax.ShapeDtypeStruct((_NC, _ACCN, _D), jnp.float32),
        scratch_types=[
            pltpu.VMEM_SHARED((_ACCN, _D), jnp.float32),
            pltpu.VMEM((3, _CHUNK), jnp.int32),
            pltpu.VMEM((3, _CHUNK), jnp.int32),
            pltpu.VMEM((_CHUNK,), jnp.int32),
            pltpu.VMEM((_CHUNK,), jnp.int32),
            pltpu.VMEM((_CHUNK,), jnp.float32),
            pltpu.VMEM((_CHUNK,), jnp.float32),
            pltpu.VMEM((_CHUNK, _D), jnp.float32),
            pltpu.VMEM((_CHUNK, _D), jnp.float32),
            pltpu.VMEM((_CHUNK, _D), jnp.float32),
            pltpu.VMEM((_CHUNK, _D), jnp.float32),
            pltpu.SemaphoreType.DMA,
            pltpu.SemaphoreType.DMA,
            pltpu.SemaphoreType.DMA,
            pltpu.SemaphoreType.DMA,
            pltpu.SemaphoreType.DMA,
            pltpu.SemaphoreType.DMA,
        ],
        compiler_params=_sc_compiler_params(),
    )
    def agg_kernel(h_hbm, edata_hbm, out_hbm,
                   acc, ebufa, ebufb, cbufa, cbufb, nbufa, nbufb,
                   gbufa, gbufb, sbufa, sbufb, sea, seb, sga, sgb, ssa, ssb):
        cid = lax.axis_index("c")
        sid = lax.axis_index("s")
        w = sid * _NC + cid

        z16 = jnp.zeros((16,), jnp.float32)

        @pl.loop(0, _CHUNK)
        def _(r):
            for d in range(_D16):
                sbufa[r, pl.ds(d * 16, 16)] = z16

        @pl.loop(0, _RPS // _CHUNK)
        def _(k):
            pltpu.sync_copy(sbufa, acc.at[pl.ds(sid * _RPS + k * _CHUNK, _CHUNK)])

        plsc.subcore_barrier()

        def eload(ci, ebuf, sem):
            return pltpu.make_async_copy(edata_hbm.at[w, ci], ebuf, sem)

        def gather(ebuf, gbuf, sem):
            return pltpu.make_async_copy(h_hbm.at[ebuf.at[0]], gbuf, sem)

        def scatter_wait(sbuf, cbuf, sem):
            pltpu.make_async_copy(sbuf, acc.at[cbuf], sem).wait()

        def scatter_start(sbuf, cbuf, sem):
            pltpu.async_copy(sbuf, acc.at[cbuf], sem, add=True)

        def stage_edge(ebuf, cbuf, nbuf):
            @pl.loop(0, _G16)
            def _(j):
                cbuf[pl.ds(j * 16, 16)] = ebuf[1, pl.ds(j * 16, 16)]
                nbuf[pl.ds(j * 16, 16)] = plsc.bitcast(
                    ebuf[2, pl.ds(j * 16, 16)], jnp.float32)

        def scale(nbuf, gbuf, sbuf):
            @pl.loop(0, _G16)
            def _(j):
                n16 = nbuf[pl.ds(j * 16, 16)]
                for t in range(16):
                    s16 = _splat(n16, t)
                    e = j * 16 + t
                    for d in range(_D16):
                        sbuf[e, pl.ds(d * 16, 16)] = gbuf[e, pl.ds(d * 16, 16)] * s16

        eload(0, ebufa, sea).start()
        eload(0, ebufa, sea).wait()
        gather(ebufa, gbufa, sga).start()
        eload(1, ebufb, seb).start()

        @pl.loop(0, _NCHUNK - 1, step=2)
        def _(ci):
            eload(ci + 1, ebufb, seb).wait()
            gather(ebufb, gbufb, sgb).start()
            gather(ebufa, gbufa, sga).wait()

            @pl.when(ci >= 2)
            def _():
                scatter_wait(sbufa, cbufa, ssa)

            stage_edge(ebufa, cbufa, nbufa)
            eload(ci + 2, ebufa, sea).start()
            scale(nbufa, gbufa, sbufa)
            scatter_start(sbufa, cbufa, ssa)

            eload(ci + 2, ebufa, sea).wait()
            gather(ebufa, gbufa, sga).start()
            gather(ebufb, gbufb, sgb).wait()

            @pl.when(ci >= 2)
            def _():
                scatter_wait(sbufb, cbufb, ssb)

            stage_edge(ebufb, cbufb, nbufb)

            @pl.when(ci + 3 < _NCHUNK)
            def _():
                eload(ci + 3, ebufb, seb).start()

            scale(nbufb, gbufb, sbufb)
            scatter_start(sbufb, cbufb, ssb)

        gather(ebufa, gbufa, sga).wait()
        scatter_wait(sbufa, cbufa, ssa)
        stage_edge(ebufa, cbufa, nbufa)
        scale(nbufa, gbufa, sbufa)
        scatter_start(sbufa, cbufa, ssa)
        scatter_wait(sbufb, cbufb, ssb)
        scatter_wait(sbufa, cbufa, ssa)

        plsc.subcore_barrier()

        pltpu.sync_copy(acc.at[pl.ds(sid * _RPS, _RPS)],
                        out_hbm.at[cid, pl.ds(sid * _RPS, _RPS)])

    return agg_kernel


_MMB = 1000


def _mm_body(x_ref, w_ref, o_ref):
    o_ref[...] = jnp.dot(x_ref[...], w_ref[...],
                         preferred_element_type=jnp.float32)


def _tc_matmul(x, w):
    return pl.pallas_call(
        _mm_body,
        grid=(_N // _MMB,),
        in_specs=[
            pl.BlockSpec((_MMB, _D), lambda i: (i, 0)),
            pl.BlockSpec((_D, _D), lambda i: (0, 0)),
        ],
        out_specs=pl.BlockSpec((_MMB, _D), lambda i: (i, 0)),
        out_shape=jax.ShapeDtypeStruct((_N, _D), jnp.float32),
    )(x, w)


_DVB = 2000


def _dinv_body(parts_ref, o1_ref, o2_ref):
    deg = jnp.sum(parts_ref[...], axis=0, keepdims=True) + 1.0
    safe = jnp.where(deg > 0, deg, 1.0)
    dv = jnp.where(deg > 0, lax.rsqrt(safe), 0.0)
    o1_ref[...] = dv
    o2_ref[...] = jnp.transpose(dv * dv)


def _tc_dinv(parts):
    return pl.pallas_call(
        _dinv_body,
        out_shape=[
            jax.ShapeDtypeStruct((1, _N), jnp.float32),
            jax.ShapeDtypeStruct((_N, 1), jnp.float32),
        ],
    )(parts)


def _epilogue(parts_ref, h_ref, d2_ref, b_ref, g_ref, be_ref, al_ref,
              apply_act):
    agg = (parts_ref[0] + parts_ref[1] + d2_ref[...] * h_ref[...]) + b_ref[...]
    hbn = agg * (g_ref[...] * _BN_SCALE) + be_ref[...]
    if apply_act:
        alpha = al_ref[0, 0]
        gelu = hbn * 0.5 * (1.0 + lax.erf(hbn * (1.0 / math.sqrt(2.0))))
        hbn = alpha * jnp.maximum(hbn, 0.0) + (1.0 - alpha) * gelu
    return hbn


def _post_body(parts_ref, h_ref, d2_ref, b_ref, g_ref, be_ref, al_ref, o_ref):
    o_ref[...] = _epilogue(parts_ref, h_ref, d2_ref, b_ref, g_ref, be_ref,
                           al_ref, False)


def _postmm_body(parts_ref, h_ref, d2_ref, b_ref, g_ref, be_ref, al_ref,
                 w_ref, o_ref):
    y = _epilogue(parts_ref, h_ref, d2_ref, b_ref, g_ref, be_ref, al_ref, True)
    o_ref[...] = jnp.dot(y, w_ref[...], preferred_element_type=jnp.float32)


_PARTS_SPEC = pl.BlockSpec((_NC, _MMB, _D), lambda i: (0, i, 0))
_ROW_SPEC = pl.BlockSpec((_MMB, _D), lambda i: (i, 0))
_VEC_SPEC = pl.BlockSpec((1, _D), lambda i: (0, 0))
_COL_SPEC = pl.BlockSpec((_MMB, 1), lambda i: (i, 0))
_SMEM_SPEC = pl.BlockSpec(memory_space=pltpu.SMEM)


def _tc_post(parts, h, d2, b, g, be, alpha):
    return pl.pallas_call(
        _post_body,
        grid=(_N // _MMB,),
        in_specs=[_PARTS_SPEC, _ROW_SPEC, _COL_SPEC,
                  _VEC_SPEC, _VEC_SPEC, _VEC_SPEC, _SMEM_SPEC],
        out_specs=_ROW_SPEC,
        out_shape=jax.ShapeDtypeStruct((_N, _D), jnp.float32),
    )(parts, h, d2, b, g, be, alpha)


def _tc_post_mm(parts, h, d2, b, g, be, alpha, w):
    return pl.pallas_call(
        _postmm_body,
        grid=(_N // _MMB,),
        in_specs=[_PARTS_SPEC, _ROW_SPEC, _COL_SPEC,
                  _VEC_SPEC, _VEC_SPEC, _VEC_SPEC, _SMEM_SPEC,
                  pl.BlockSpec((_D, _D), lambda i: (0, 0))],
        out_specs=_ROW_SPEC,
        out_shape=jax.ShapeDtypeStruct((_N, _D), jnp.float32),
    )(parts, h, d2, b, g, be, alpha, w)


def kernel(x, edge_index, edge_weight, W0, b0, W1, b1,
           gamma0, beta0, gamma1, beta1, act_params):
    row2 = edge_index[0].reshape(_NW, _NCHUNK, _CHUNK)
    col2 = edge_index[1].reshape(_NW, _NCHUNK, _CHUNK)
    ew2 = edge_weight.reshape(_NW, _NCHUNK, _CHUNK)

    deg_parts = _build_sc_deg()(col2, ew2).reshape(_NW, _N)
    dinv_row, dinv2_col = _tc_dinv(deg_parts)
    dinv = dinv_row.reshape(_N)
    norm2 = _build_sc_norm()(row2, col2, ew2, dinv)
    normi = lax.bitcast_convert_type(norm2, jnp.int32)
    edata = jnp.stack([row2, col2, normi], axis=2)

    alpha = jax.nn.sigmoid(act_params[0]).reshape(1, 1)

    b0r = b0.reshape(1, _D)
    g0r = gamma0.reshape(1, _D)
    be0r = beta0.reshape(1, _D)
    b1r = b1.reshape(1, _D)
    g1r = gamma1.reshape(1, _D)
    be1r = beta1.reshape(1, _D)

    agg = _build_sc_agg()

    h0 = _tc_matmul(x, W0)
    parts0 = agg(h0, edata)
    h1 = _tc_post_mm(parts0, h0, dinv2_col, b0r, g0r, be0r, alpha, W1)
    parts1 = agg(h1, edata)
    return _tc_post(parts1, h1, dinv2_col, b1r, g1r, be1r, alpha)


# speedup vs baseline: 25.5078x; 1.0063x over previous
import dataclasses
import functools
import math

import jax
import jax.numpy as jnp
from jax import lax
from jax.experimental import pallas as pl
from jax.experimental.pallas import tpu as pltpu
from jax.experimental.pallas import tpu_sc as plsc

_N = 10000
_E = 320000
_D = 128

_NC = 2
_NS = 16
_NW = _NC * _NS
_EPW = _E // _NW
_CHUNK = 80
_NCHUNK = _EPW // _CHUNK
_ACCN = 10240
_RPS = _ACCN // _NS
_ZR = 64
_G16 = _CHUNK // 16
_D16 = _D // 16

_BN_SCALE = 1.0 / math.sqrt(1.0 + 1e-5)


def _worker_id():
    return lax.axis_index("s") * _NC + lax.axis_index("c")


def _sc_compiler_params():
    cp = pltpu.CompilerParams()
    if "needs_layout_passes" in pltpu.CompilerParams.__dataclass_fields__:
        cp = dataclasses.replace(cp, needs_layout_passes=False)
    return cp


@functools.lru_cache(maxsize=None)
def _build_sc_deg():
    mesh = plsc.VectorSubcoreMesh(core_axis_name="c", subcore_axis_name="s")

    @functools.partial(
        pl.kernel,
        mesh=mesh,
        out_type=jax.ShapeDtypeStruct((_NW, 1, _N), jnp.float32),
        scratch_types=[
            pltpu.VMEM((1, _N), jnp.float32),
            pltpu.VMEM((_NCHUNK, _CHUNK), jnp.int32),
            pltpu.VMEM((_NCHUNK, _CHUNK), jnp.float32),
        ],
        compiler_params=_sc_compiler_params(),
    )
    def deg_kernel(col_hbm, ew_hbm, out_hbm, deg_v, col_v, ew_v):
        w = _worker_id()
        pltpu.sync_copy(col_hbm.at[w], col_v)
        pltpu.sync_copy(ew_hbm.at[w], ew_v)

        z16 = jnp.zeros((16,), jnp.float32)
        z16i = jnp.zeros((16,), jnp.int32)

        @pl.loop(0, _N // 16)
        def _(g):
            deg_v[0, pl.ds(g * 16, 16)] = z16

        @pl.loop(0, _NCHUNK)
        def _(ci):
            for j in range(_G16):
                idx = col_v[ci, pl.ds(j * 16, 16)]
                val = ew_v[ci, pl.ds(j * 16, 16)]
                plsc.addupdate_scatter(deg_v, [z16i, idx], val)

        pltpu.sync_copy(deg_v, out_hbm.at[w])

    return deg_kernel


@functools.lru_cache(maxsize=None)
def _build_sc_norm():
    mesh = plsc.VectorSubcoreMesh(core_axis_name="c", subcore_axis_name="s")

    @functools.partial(
        pl.kernel,
        mesh=mesh,
        out_type=jax.ShapeDtypeStruct((_NW, _NCHUNK, _CHUNK), jnp.float32),
        scratch_types=[
            pltpu.VMEM((_NCHUNK, _CHUNK), jnp.int32),
            pltpu.VMEM((_NCHUNK, _CHUNK), jnp.int32),
            pltpu.VMEM((_NCHUNK, _CHUNK), jnp.float32),
            pltpu.VMEM((_N,), jnp.float32),
        ],
        compiler_params=_sc_compiler_params(),
    )
    def norm_kernel(row_hbm, col_hbm, ew_hbm, dinv_hbm, out_hbm,
                    row_v, col_v, ew_v, dinv_v):
        w = _worker_id()
        pltpu.sync_copy(dinv_hbm, dinv_v)
        pltpu.sync_copy(row_hbm.at[w], row_v)
        pltpu.sync_copy(col_hbm.at[w], col_v)
        pltpu.sync_copy(ew_hbm.at[w], ew_v)

        @pl.loop(0, _NCHUNK)
        def _(ci):
            for j in range(_G16):
                r16 = row_v[ci, pl.ds(j * 16, 16)]
                c16 = col_v[ci, pl.ds(j * 16, 16)]
                w16 = ew_v[ci, pl.ds(j * 16, 16)]
                dr = plsc.load_gather(dinv_v, [r16])
                dc = plsc.load_gather(dinv_v, [c16])
                ew_v[ci, pl.ds(j * 16, 16)] = dr * w16 * dc

        pltpu.sync_copy(ew_v, out_hbm.at[w])

    return norm_kernel


_SPLAT_DNUMS = lax.GatherDimensionNumbers(
    offset_dims=(), collapsed_slice_dims=(0,), start_index_map=(0,))


def _splat(v16, t):
    idx = jnp.full((16, 1), t, jnp.int32)
    return lax.gather(v16, idx, _SPLAT_DNUMS, (1,),
                      mode=lax.GatherScatterMode.PROMISE_IN_BOUNDS)


@functools.lru_cache(maxsize=None)
def _build_sc_agg():
    mesh = plsc.VectorSubcoreMesh(core_axis_name="c", subcore_axis_name="s")

    @functools.partial(
        pl.kernel,
        mesh=mesh,
        out_type=jax.ShapeDtypeStruct((_NC, _ACCN, _D), jnp.float32),
        scratch_types=[
            pltpu.VMEM_SHARED((_ACCN, _D), jnp.float32),
            pltpu.VMEM((3, _CHUNK), jnp.int32),
            pltpu.VMEM((3, _CHUNK), jnp.int32),
            pltpu.VMEM((_CHUNK,), jnp.int32),
            pltpu.VMEM((_CHUNK,), jnp.int32),
            pltpu.VMEM((_CHUNK,), jnp.float32),
            pltpu.VMEM((_CHUNK,), jnp.float32),
            pltpu.VMEM((_CHUNK, _D), jnp.float32),
            pltpu.VMEM((_CHUNK, _D), jnp.float32),
            pltpu.VMEM((_CHUNK, _D), jnp.float32),
            pltpu.VMEM((_CHUNK, _D), jnp.float32),
            pltpu.SemaphoreType.DMA,
            pltpu.SemaphoreType.DMA,
            pltpu.SemaphoreType.DMA,
            pltpu.SemaphoreType.DMA,
            pltpu.SemaphoreType.DMA,
            pltpu.SemaphoreType.DMA,
        ],
        compiler_params=_sc_compiler_params(),
    )
    def agg_kernel(h_hbm, edata_hbm, out_hbm,
                   acc, ebufa, ebufb, cbufa, cbufb, nbufa, nbufb,
                   gbufa, gbufb, sbufa, sbufb, sea, seb, sga, sgb, ssa, ssb):
        cid = lax.axis_index("c")
        sid = lax.axis_index("s")
        w = sid * _NC + cid

        z16 = jnp.zeros((16,), jnp.float32)

        def eload(ci, ebuf, sem):
            return pltpu.make_async_copy(edata_hbm.at[w, ci], ebuf, sem)

        _H = _CHUNK // 2

        def gather_descs(ebuf, gbuf, sem):
            return (
                pltpu.make_async_copy(
                    h_hbm.at[ebuf.at[0, pl.ds(0, _H)]],
                    gbuf.at[pl.ds(0, _H)], sem),
                pltpu.make_async_copy(
                    h_hbm.at[ebuf.at[0, pl.ds(_H, _H)]],
                    gbuf.at[pl.ds(_H, _H)], sem),
            )

        class _GatherPair:
            def __init__(self, descs):
                self._descs = descs

            def start(self):
                for d in self._descs:
                    d.start()

            def wait(self):
                for d in self._descs:
                    d.wait()

        def gather(ebuf, gbuf, sem):
            return _GatherPair(gather_descs(ebuf, gbuf, sem))

        def scatter_wait(sbuf, cbuf, sem):
            pltpu.make_async_copy(sbuf, acc.at[cbuf], sem).wait()

        def scatter_start(sbuf, cbuf, sem):
            pltpu.async_copy(sbuf, acc.at[cbuf], sem, add=True)

        def stage_edge(ebuf, cbuf, nbuf):
            @pl.loop(0, _G16)
            def _(j):
                cbuf[pl.ds(j * 16, 16)] = ebuf[1, pl.ds(j * 16, 16)]
                nbuf[pl.ds(j * 16, 16)] = plsc.bitcast(
                    ebuf[2, pl.ds(j * 16, 16)], jnp.float32)

        def scale(nbuf, gbuf, sbuf):
            @pl.loop(0, _G16)
            def _(j):
                n16 = nbuf[pl.ds(j * 16, 16)]
                for t in range(16):
                    s16 = _splat(n16, t)
                    e = j * 16 + t
                    for d in range(_D16):
                        sbuf[e, pl.ds(d * 16, 16)] = gbuf[e, pl.ds(d * 16, 16)] * s16

        eload(0, ebufa, sea).start()

        @pl.loop(0, _CHUNK)
        def _(r):
            for d in range(_D16):
                sbufa[r, pl.ds(d * 16, 16)] = z16

        eload(0, ebufa, sea).wait()
        gather(ebufa, gbufa, sga).start()
        eload(1, ebufb, seb).start()

        @pl.loop(0, _RPS // _CHUNK)
        def _(k):
            pltpu.sync_copy(sbufa, acc.at[pl.ds(sid * _RPS + k * _CHUNK, _CHUNK)])

        plsc.subcore_barrier()

        @pl.loop(0, _NCHUNK - 1, step=2)
        def _(ci):
            eload(ci + 1, ebufb, seb).wait()
            gather(ebufb, gbufb, sgb).start()
            gather(ebufa, gbufa, sga).wait()

            @pl.when(ci >= 2)
            def _():
                scatter_wait(sbufa, cbufa, ssa)

            stage_edge(ebufa, cbufa, nbufa)
            eload(ci + 2, ebufa, sea).start()
            scale(nbufa, gbufa, sbufa)
            scatter_start(sbufa, cbufa, ssa)

            eload(ci + 2, ebufa, sea).wait()
            gather(ebufa, gbufa, sga).start()
            gather(ebufb, gbufb, sgb).wait()

            @pl.when(ci >= 2)
            def _():
                scatter_wait(sbufb, cbufb, ssb)

            stage_edge(ebufb, cbufb, nbufb)

            @pl.when(ci + 3 < _NCHUNK)
            def _():
                eload(ci + 3, ebufb, seb).start()

            scale(nbufb, gbufb, sbufb)
            scatter_start(sbufb, cbufb, ssb)

        gather(ebufa, gbufa, sga).wait()
        scatter_wait(sbufa, cbufa, ssa)
        stage_edge(ebufa, cbufa, nbufa)
        scale(nbufa, gbufa, sbufa)
        scatter_start(sbufa, cbufa, ssa)
        scatter_wait(sbufb, cbufb, ssb)
        scatter_wait(sbufa, cbufa, ssa)

        plsc.subcore_barrier()

        pltpu.sync_copy(acc.at[pl.ds(sid * _RPS, _RPS)],
                        out_hbm.at[cid, pl.ds(sid * _RPS, _RPS)])

    return agg_kernel


_MMB = 1000


def _mm_body(x_ref, w_ref, o_ref):
    o_ref[...] = jnp.dot(x_ref[...], w_ref[...],
                         preferred_element_type=jnp.float32)


def _tc_matmul(x, w):
    return pl.pallas_call(
        _mm_body,
        grid=(_N // _MMB,),
        in_specs=[
            pl.BlockSpec((_MMB, _D), lambda i: (i, 0)),
            pl.BlockSpec((_D, _D), lambda i: (0, 0)),
        ],
        out_specs=pl.BlockSpec((_MMB, _D), lambda i: (i, 0)),
        out_shape=jax.ShapeDtypeStruct((_N, _D), jnp.float32),
    )(x, w)


_DVB = 2000


def _dinv_body(parts_ref, o1_ref, o2_ref):
    deg = jnp.sum(parts_ref[...], axis=0, keepdims=True) + 1.0
    safe = jnp.where(deg > 0, deg, 1.0)
    dv = jnp.where(deg > 0, lax.rsqrt(safe), 0.0)
    o1_ref[...] = dv
    o2_ref[...] = jnp.transpose(dv * dv)


def _tc_dinv(parts):
    return pl.pallas_call(
        _dinv_body,
        out_shape=[
            jax.ShapeDtypeStruct((1, _N), jnp.float32),
            jax.ShapeDtypeStruct((_N, 1), jnp.float32),
        ],
    )(parts)


def _epilogue(parts_ref, h_ref, d2_ref, b_ref, g_ref, be_ref, al_ref,
              apply_act):
    agg = (parts_ref[0] + parts_ref[1] + d2_ref[...] * h_ref[...]) + b_ref[...]
    hbn = agg * (g_ref[...] * _BN_SCALE) + be_ref[...]
    if apply_act:
        alpha = al_ref[0, 0]
        gelu = hbn * 0.5 * (1.0 + lax.erf(hbn * (1.0 / math.sqrt(2.0))))
        hbn = alpha * jnp.maximum(hbn, 0.0) + (1.0 - alpha) * gelu
    return hbn


def _post_body(parts_ref, h_ref, d2_ref, b_ref, g_ref, be_ref, al_ref, o_ref):
    o_ref[...] = _epilogue(parts_ref, h_ref, d2_ref, b_ref, g_ref, be_ref,
                           al_ref, False)


def _postmm_body(parts_ref, h_ref, d2_ref, b_ref, g_ref, be_ref, al_ref,
                 w_ref, o_ref):
    y = _epilogue(parts_ref, h_ref, d2_ref, b_ref, g_ref, be_ref, al_ref, True)
    o_ref[...] = jnp.dot(y, w_ref[...], preferred_element_type=jnp.float32)


_PARTS_SPEC = pl.BlockSpec((_NC, _MMB, _D), lambda i: (0, i, 0))
_ROW_SPEC = pl.BlockSpec((_MMB, _D), lambda i: (i, 0))
_VEC_SPEC = pl.BlockSpec((1, _D), lambda i: (0, 0))
_COL_SPEC = pl.BlockSpec((_MMB, 1), lambda i: (i, 0))
_SMEM_SPEC = pl.BlockSpec(memory_space=pltpu.SMEM)


def _tc_post(parts, h, d2, b, g, be, alpha):
    return pl.pallas_call(
        _post_body,
        grid=(_N // _MMB,),
        in_specs=[_PARTS_SPEC, _ROW_SPEC, _COL_SPEC,
                  _VEC_SPEC, _VEC_SPEC, _VEC_SPEC, _SMEM_SPEC],
        out_specs=_ROW_SPEC,
        out_shape=jax.ShapeDtypeStruct((_N, _D), jnp.float32),
    )(parts, h, d2, b, g, be, alpha)


def _tc_post_mm(parts, h, d2, b, g, be, alpha, w):
    return pl.pallas_call(
        _postmm_body,
        grid=(_N // _MMB,),
        in_specs=[_PARTS_SPEC, _ROW_SPEC, _COL_SPEC,
                  _VEC_SPEC, _VEC_SPEC, _VEC_SPEC, _SMEM_SPEC,
                  pl.BlockSpec((_D, _D), lambda i: (0, 0))],
        out_specs=_ROW_SPEC,
        out_shape=jax.ShapeDtypeStruct((_N, _D), jnp.float32),
    )(parts, h, d2, b, g, be, alpha, w)


def kernel(x, edge_index, edge_weight, W0, b0, W1, b1,
           gamma0, beta0, gamma1, beta1, act_params):
    row2 = edge_index[0].reshape(_NW, _NCHUNK, _CHUNK)
    col2 = edge_index[1].reshape(_NW, _NCHUNK, _CHUNK)
    ew2 = edge_weight.reshape(_NW, _NCHUNK, _CHUNK)

    deg_parts = _build_sc_deg()(col2, ew2).reshape(_NW, _N)
    dinv_row, dinv2_col = _tc_dinv(deg_parts)
    dinv = dinv_row.reshape(_N)
    norm2 = _build_sc_norm()(row2, col2, ew2, dinv)
    normi = lax.bitcast_convert_type(norm2, jnp.int32)
    edata = jnp.stack([row2, col2, normi], axis=2)

    alpha = jax.nn.sigmoid(act_params[0]).reshape(1, 1)

    b0r = b0.reshape(1, _D)
    g0r = gamma0.reshape(1, _D)
    be0r = beta0.reshape(1, _D)
    b1r = b1.reshape(1, _D)
    g1r = gamma1.reshape(1, _D)
    be1r = beta1.reshape(1, _D)

    agg = _build_sc_agg()

    h0 = _tc_matmul(x, W0)
    parts0 = agg(h0, edata)
    h1 = _tc_post_mm(parts0, h0, dinv2_col, b0r, g0r, be0r, alpha, W1)
    parts1 = agg(h1, edata)
    return _tc_post(parts1, h1, dinv2_col, b1r, g1r, be1r, alpha)
